# baseline (speedup 1.0000x reference)
import functools

import jax
import jax.numpy as jnp
from jax import lax
from jax.experimental import pallas as pl
from jax.experimental.pallas import tpu as pltpu
from jax.experimental.pallas import tpu_sc as plsc

VOCAB = 50368
HIDDEN = 768
EPS = 1e-05

N_TOKENS = 4 * 8192
NUM_CORES = 2
NUM_SUBCORES = 16
NUM_WORKERS = NUM_CORES * NUM_SUBCORES
PER_WORKER = N_TOKENS // NUM_WORKERS
CHUNK = 128
NUM_CHUNKS = PER_WORKER // CHUNK
LANES = 16
NVEC = HIDDEN // LANES


def _lane_sum(v):
    iota = lax.iota(jnp.int32, LANES)
    dnums = lax.GatherDimensionNumbers(
        offset_dims=(), collapsed_slice_dims=(0,), start_index_map=(0,))
    for sh in (1, 2, 4, 8):
        perm = lax.gather(v, (iota ^ sh)[:, None], dnums, slice_sizes=(1,),
                          mode=lax.GatherScatterMode.PROMISE_IN_BOUNDS)
        v = v + perm
    return v


def _body(ids_hbm, table_hbm, w_hbm, out_hbm, idx_v, rows_v, w_v, sem):
    wid = lax.axis_index("s") * NUM_CORES + lax.axis_index("c")
    base = wid * PER_WORKER

    pltpu.sync_copy(w_hbm, w_v)

    def chunk_body(ci, carry):
        off = base + ci * CHUNK
        pltpu.sync_copy(ids_hbm.at[pl.ds(off, CHUNK)], idx_v)
        pltpu.async_copy(table_hbm.at[idx_v], rows_v, sem).wait()

        def row_body(r, c2):
            acc = jnp.zeros((LANES,), jnp.float32)
            acc2 = jnp.zeros((LANES,), jnp.float32)
            for j in range(NVEC):
                x = rows_v[r, pl.ds(LANES * j, LANES)]
                acc = acc + x
                acc2 = acc2 + x * x
            s = _lane_sum(acc)
            s2 = _lane_sum(acc2)
            mean = s * (1.0 / HIDDEN)
            var = s2 * (1.0 / HIDDEN) - mean * mean
            y = var + EPS
            i = plsc.bitcast(y, jnp.int32)
            i = 0x5F3759DF - (i >> 1)
            g = plsc.bitcast(i, jnp.float32)
            g = g * (1.5 - 0.5 * y * g * g)
            g = g * (1.5 - 0.5 * y * g * g)
            g = g * (1.5 - 0.5 * y * g * g)
            a = g
            b = -mean * g
            for j in range(NVEC):
                sl = pl.ds(LANES * j, LANES)
                x = rows_v[r, sl]
                rows_v[r, sl] = (x * a + b) * w_v[sl]
            return c2

        lax.fori_loop(0, CHUNK, row_body, 0)
        pltpu.sync_copy(rows_v, out_hbm.at[pl.ds(off, CHUNK)])
        return carry

    lax.fori_loop(0, NUM_CHUNKS, chunk_body, 0)


_sc_call = functools.partial(
    pl.kernel,
    mesh=plsc.VectorSubcoreMesh(core_axis_name="c", subcore_axis_name="s"),
    out_type=jax.ShapeDtypeStruct((N_TOKENS, HIDDEN), jnp.float32),
    scratch_types=[
        pltpu.VMEM((CHUNK,), jnp.int32),
        pltpu.VMEM((CHUNK, HIDDEN), jnp.float32),
        pltpu.VMEM((HIDDEN,), jnp.float32),
        pltpu.SemaphoreType.DMA,
    ],
    compiler_params=pltpu.CompilerParams(needs_layout_passes=False),
)(_body)


@jax.jit
def kernel(input_ids, tok_embeddings, norm_weight):
    ids = input_ids.reshape(-1).astype(jnp.int32)
    out = _sc_call(ids, tok_embeddings, norm_weight)
    return out.reshape(input_ids.shape + (HIDDEN,))
